# Initial kernel scaffold; baseline (speedup 1.0000x reference)
import jax
import jax.numpy as jnp
from jax.experimental import pallas as pl


def kernel(m, rbf, sbf, triplet_index, W_rbf1, W_rbf2, W_sbf1, W_sbf2, W_ji, b_ji, W_kj, b_kj, W_down, W_up, Wb1, bb1, Wb2, bb2, Wf, bf, Wa1, ba1, Wa2, ba2):
    raise NotImplementedError("write your pallas kernel here")


# speedup vs baseline: 1.3065x; 1.3065x over previous
import functools

import jax
import jax.numpy as jnp
from jax import lax
from jax.experimental import pallas as pl
from jax.experimental.pallas import tpu as pltpu
from jax.experimental.pallas import tpu_sc as plsc

E = 320000
T = 1280000
EMB = 128
INT = 64

_NC, _NS = 2, 16
_R = 22912
_NPASS = 14
_EPAD = _R * _NPASS
_SHARE = _R // _NS
_ZROWS = 128
_CHUNK = T // _NS
_BSC = 800
_NBLK = _CHUNK // _BSC
_VPB = _BSC // 16
_GBUF = 128
_GB = 112
_ACC = _R + 8


def _edge_body(m_ref, rbf_ref, wr1, wr2, wkj, bkj, wdown, out_ref):
    rbf_t = (rbf_ref[...] @ wr1[...]) @ wr2[...]
    xkj = m_ref[...] @ wkj[...] + bkj[...]
    out_ref[...] = (xkj * rbf_t) @ wdown[...]


def _sbf_body(sbf_ref, ws1, ws2, out_ref):
    out_ref[...] = (sbf_ref[...] @ ws1[...]) @ ws2[...]


def _epi_body(m_ref, magg_ref, wup, wji, bji, wb1, bb1, wb2, bb2, wf, bf_,
              wa1, ba1, wa2, ba2, out_ref):
    m = m_ref[...]
    mu = magg_ref[...] @ wup[...] + (m @ wji[...] + bji[...])
    h = (mu @ wb1[...] + bb1[...]) @ wb2[...] + bb2[...]
    mu = mu + h
    mu = mu @ wf[...] + bf_[...]
    mo = m + mu
    a1 = wa1[...]
    a2 = wa2[...]
    c1 = ba1[...]
    c2 = ba2[...]
    for i in range(2):
        h = (mo @ a1[i] + c1[i]) @ a2[i] + c2[i]
        mo = mo + h
    out_ref[...] = mo


def _full(shape):
    nd = len(shape)
    return pl.BlockSpec(shape, lambda i: (0,) * nd)


def _edge_call(m, rbf, wr1, wr2, wkj, bkj, wdown):
    be = 2560
    grid = (E // be,)
    return pl.pallas_call(
        _edge_body,
        grid=grid,
        in_specs=[
            pl.BlockSpec((be, EMB), lambda i: (i, 0)),
            pl.BlockSpec((be, 6), lambda i: (i, 0)),
            _full(wr1.shape), _full(wr2.shape), _full(wkj.shape),
            _full(bkj.shape), _full(wdown.shape),
        ],
        out_specs=pl.BlockSpec((be, INT), lambda i: (i, 0)),
        out_shape=jax.ShapeDtypeStruct((E, INT), jnp.float32),
    )(m, rbf, wr1, wr2, wkj, bkj, wdown)


def _sbf_call(sbf, ws1, ws2):
    bt = 5120
    grid = (T // bt,)
    return pl.pallas_call(
        _sbf_body,
        grid=grid,
        in_specs=[
            pl.BlockSpec((bt, 42), lambda i: (i, 0)),
            _full(ws1.shape), _full(ws2.shape),
        ],
        out_specs=pl.BlockSpec((bt, INT), lambda i: (i, 0)),
        out_shape=jax.ShapeDtypeStruct((T, INT), jnp.float32),
    )(sbf, ws1, ws2)


def _epi_call(m, magg, wup, wji, bji, wb1, bb1, wb2, bb2, wf, bf_, wa1, ba1, wa2, ba2):
    be = 2560
    grid = (E // be,)
    args = (wup, wji, bji, wb1, bb1, wb2, bb2, wf, bf_, wa1, ba1, wa2, ba2)
    return pl.pallas_call(
        _epi_body,
        grid=grid,
        in_specs=[
            pl.BlockSpec((be, EMB), lambda i: (i, 0)),
            pl.BlockSpec((be, INT), lambda i: (i, 0)),
        ] + [_full(a.shape) for a in args],
        out_specs=pl.BlockSpec((be, EMB), lambda i: (i, 0)),
        out_shape=jax.ShapeDtypeStruct((E, EMB), jnp.float32),
    )(m, magg, *args)


def _sc_body(src_h, dst_h, xkj_h, sbft_h, out_h,
             dstblk, srcblk, slist, tlist, dlist, xrows, srows, zbuf, acc,
             sem1, sem2):
    cid = lax.axis_index("c")
    sid = lax.axis_index("s")
    zf = jnp.zeros((16,), jnp.float32)
    zi = jnp.zeros((16,), jnp.int32)
    dummy = jnp.full((16,), _R, jnp.int32)

    def _zb(i, c):
        for q in range(INT // 16):
            zbuf[i, pl.ds(q * 16, 16)] = zf
        return c
    lax.fori_loop(0, _ZROWS, _zb, 0)

    def _reset_lists():
        for j in range(_GBUF // 16):
            sl = pl.ds(j * 16, 16)
            slist[sl] = zi
            tlist[sl] = zi
            dlist[sl] = dummy
    _reset_lists()

    def _drain():
        cp1 = pltpu.make_async_copy(xkj_h.at[slist], xrows, sem1)
        cp2 = pltpu.make_async_copy(sbft_h.at[tlist], srows, sem2)
        cp1.start()
        cp2.start()
        cp1.wait()
        cp2.wait()

        def _mul(r, c):
            for q in range(INT // 16):
                sl = pl.ds(q * 16, 16)
                srows[r, sl] = srows[r, sl] * xrows[r, sl]
            return c
        lax.fori_loop(0, _GBUF, _mul, 0)
        pltpu.sync_copy(srows, acc.at[dlist], add=True)
        _reset_lists()
        return 0

    chunk0 = sid * _CHUNK

    def _do_pass(p):
        lo = p * _R
        r0 = sid * _SHARE
        off = 0
        while off < _SHARE:
            n = min(_ZROWS, _SHARE - off)
            zsrc = zbuf if n == _ZROWS else zbuf.at[pl.ds(0, n)]
            pltpu.sync_copy(zsrc, acc.at[pl.ds(r0 + off, n)])
            off += n
        plsc.subcore_barrier()

        def _blk(b, cnt):
            t0 = chunk0 + b * _BSC
            pltpu.sync_copy(dst_h.at[pl.ds(t0, _BSC)], dstblk)
            pltpu.sync_copy(src_h.at[pl.ds(t0, _BSC)], srcblk)

            def _v(v, cnt):
                sl = pl.ds(v * 16, 16)
                d = dstblk[sl]
                s = srcblk[sl]
                dl = d - lo
                msk = (dl >= 0) & (dl < _R)
                mi = msk.astype(jnp.int32)
                tid = t0 + v * 16 + lax.iota(jnp.int32, 16)
                pos = jnp.where(msk, cnt + plsc.cumsum(mi) - 1, _GBUF - 1)
                dl = jnp.where(msk, dl, _R)
                plsc.store_scatter(slist, [pos], s)
                plsc.store_scatter(tlist, [pos], tid)
                plsc.store_scatter(dlist, [pos], dl)
                cnt = cnt + jnp.sum(mi)
                return lax.cond(cnt >= _GB, _drain, lambda: cnt)
            return lax.fori_loop(0, _VPB, _v, cnt)

        cnt = lax.fori_loop(0, _NBLK, _blk, 0)
        lax.cond(cnt > 0, _drain, lambda: 0)
        plsc.subcore_barrier()
        pltpu.sync_copy(acc.at[pl.ds(r0, _SHARE)], out_h.at[pl.ds(lo + r0, _SHARE)])
        plsc.subcore_barrier()

    def _pk(k, c):
        p = k * _NC + cid

        @pl.when(p < _NPASS)
        def _():
            _do_pass(p)
        return c
    lax.fori_loop(0, _NPASS // _NC, _pk, 0)


def _sc_segsum(src, dst, xkj, sbft):
    mesh = plsc.VectorSubcoreMesh(core_axis_name="c", subcore_axis_name="s")
    fn = functools.partial(
        pl.kernel,
        out_type=jax.ShapeDtypeStruct((_EPAD, INT), jnp.float32),
        mesh=mesh,
        scratch_types=[
            pltpu.VMEM((_BSC,), jnp.int32),
            pltpu.VMEM((_BSC,), jnp.int32),
            pltpu.VMEM((_GBUF,), jnp.int32),
            pltpu.VMEM((_GBUF,), jnp.int32),
            pltpu.VMEM((_GBUF,), jnp.int32),
            pltpu.VMEM((_GBUF, INT), jnp.float32),
            pltpu.VMEM((_GBUF, INT), jnp.float32),
            pltpu.VMEM((_ZROWS, INT), jnp.float32),
            pltpu.VMEM_SHARED((_ACC, INT), jnp.float32),
            pltpu.SemaphoreType.DMA,
            pltpu.SemaphoreType.DMA,
        ],
        compiler_params=pltpu.CompilerParams(
            needs_layout_passes=False,
            use_tc_tiling_on_sc=False,
        ),
    )(_sc_body)
    return fn(src, dst, xkj, sbft)


def kernel(m, rbf, sbf, triplet_index, W_rbf1, W_rbf2, W_sbf1, W_sbf2,
           W_ji, b_ji, W_kj, b_kj, W_down, W_up, Wb1, bb1, Wb2, bb2, Wf, bf,
           Wa1, ba1, Wa2, ba2):
    xkj = _edge_call(m, rbf, W_rbf1, W_rbf2, W_kj, b_kj, W_down)
    sbft = _sbf_call(sbf, W_sbf1, W_sbf2)
    src = triplet_index[0]
    dst = triplet_index[1]
    magg = _sc_segsum(src, dst, xkj, sbft)[:E]
    return _epi_call(m, magg, W_up, W_ji, b_ji, Wb1, bb1, Wb2, bb2, Wf, bf,
                     Wa1, ba1, Wa2, ba2)


# speedup vs baseline: 1.3080x; 1.0011x over previous
import functools

import jax
import jax.numpy as jnp
from jax import lax
from jax.experimental import pallas as pl
from jax.experimental.pallas import tpu as pltpu
from jax.experimental.pallas import tpu_sc as plsc

E = 320000
T = 1280000
EMB = 128
INT = 64

_NC, _NS = 2, 16
_R = 22912
_NPASS = 14
_EPAD = _R * _NPASS
_SHARE = _R // _NS
_ZROWS = 128
_CHUNK = T // _NS
_BSC = 800
_NBLK = _CHUNK // _BSC
_VPB = _BSC // 16
_GBUF = 128
_GB = 112
_ACC = _R + 8


def _edge_body(m_ref, rbf_ref, wr1, wr2, wkj, bkj, wdown, out_ref):
    rbf_t = (rbf_ref[...] @ wr1[...]) @ wr2[...]
    xkj = m_ref[...] @ wkj[...] + bkj[...]
    out_ref[...] = (xkj * rbf_t) @ wdown[...]


def _sbf_body(sbf_ref, ws1, ws2, out_ref):
    out_ref[...] = (sbf_ref[...] @ ws1[...]) @ ws2[...]


def _epi_body(m_ref, magg_ref, wup, wji, bji, wb1, bb1, wb2, bb2, wf, bf_,
              wa1, ba1, wa2, ba2, out_ref):
    m = m_ref[...]
    mu = magg_ref[...] @ wup[...] + (m @ wji[...] + bji[...])
    h = (mu @ wb1[...] + bb1[...]) @ wb2[...] + bb2[...]
    mu = mu + h
    mu = mu @ wf[...] + bf_[...]
    mo = m + mu
    a1 = wa1[...]
    a2 = wa2[...]
    c1 = ba1[...]
    c2 = ba2[...]
    for i in range(2):
        h = (mo @ a1[i] + c1[i]) @ a2[i] + c2[i]
        mo = mo + h
    out_ref[...] = mo


def _full(shape):
    nd = len(shape)
    return pl.BlockSpec(shape, lambda i: (0,) * nd)


def _edge_call(m, rbf, wr1, wr2, wkj, bkj, wdown):
    be = 2560
    grid = (E // be,)
    return pl.pallas_call(
        _edge_body,
        grid=grid,
        in_specs=[
            pl.BlockSpec((be, EMB), lambda i: (i, 0)),
            pl.BlockSpec((be, 6), lambda i: (i, 0)),
            _full(wr1.shape), _full(wr2.shape), _full(wkj.shape),
            _full(bkj.shape), _full(wdown.shape),
        ],
        out_specs=pl.BlockSpec((be, INT), lambda i: (i, 0)),
        out_shape=jax.ShapeDtypeStruct((E, INT), jnp.float32),
    )(m, rbf, wr1, wr2, wkj, bkj, wdown)


def _sbf_call(sbf, ws1, ws2):
    bt = 5120
    grid = (T // bt,)
    return pl.pallas_call(
        _sbf_body,
        grid=grid,
        in_specs=[
            pl.BlockSpec((bt, 42), lambda i: (i, 0)),
            _full(ws1.shape), _full(ws2.shape),
        ],
        out_specs=pl.BlockSpec((bt, INT), lambda i: (i, 0)),
        out_shape=jax.ShapeDtypeStruct((T, INT), jnp.float32),
    )(sbf, ws1, ws2)


def _epi_call(m, magg, wup, wji, bji, wb1, bb1, wb2, bb2, wf, bf_, wa1, ba1, wa2, ba2):
    be = 2560
    grid = (E // be,)
    args = (wup, wji, bji, wb1, bb1, wb2, bb2, wf, bf_, wa1, ba1, wa2, ba2)
    return pl.pallas_call(
        _epi_body,
        grid=grid,
        in_specs=[
            pl.BlockSpec((be, EMB), lambda i: (i, 0)),
            pl.BlockSpec((be, INT), lambda i: (i, 0)),
        ] + [_full(a.shape) for a in args],
        out_specs=pl.BlockSpec((be, EMB), lambda i: (i, 0)),
        out_shape=jax.ShapeDtypeStruct((E, EMB), jnp.float32),
    )(m, magg, *args)


def _sc_body(src_h, dst_h, xkj_h, sbft_h, out_h,
             dstblk, srcblk, slist, tlist, dlist, sslist, stlist, sdlist,
             xrows, srows, zbuf, acc, sem1, sem2):
    cid = lax.axis_index("c")
    sid = lax.axis_index("s")
    zf = jnp.zeros((16,), jnp.float32)
    zi = jnp.zeros((16,), jnp.int32)
    dummy = jnp.full((16,), _R, jnp.int32)

    def _zb(i, c):
        for q in range(INT // 16):
            zbuf[i, pl.ds(q * 16, 16)] = zf
        return c
    lax.fori_loop(0, _ZROWS, _zb, 0)

    def _reset_lists():
        for j in range(_GBUF // 16):
            sl = pl.ds(j * 16, 16)
            slist[sl] = zi
            tlist[sl] = zi
            dlist[sl] = dummy
    _reset_lists()

    def _gathers():
        cp1 = pltpu.make_async_copy(xkj_h.at[sslist], xrows, sem1)
        cp2 = pltpu.make_async_copy(sbft_h.at[stlist], srows, sem2)
        return cp1, cp2

    def _flush_pending():
        cp1, cp2 = _gathers()
        cp1.wait()
        cp2.wait()

        def _mul(r, c):
            for q in range(INT // 16):
                sl = pl.ds(q * 16, 16)
                srows[r, sl] = srows[r, sl] * xrows[r, sl]
            return c
        lax.fori_loop(0, _GBUF, _mul, 0)
        pltpu.sync_copy(srows, acc.at[sdlist], add=True)

    def _fill(pend):
        @pl.when(pend == 1)
        def _():
            _flush_pending()
        for j in range(_GBUF // 16):
            sl = pl.ds(j * 16, 16)
            sslist[sl] = slist[sl]
            stlist[sl] = tlist[sl]
            sdlist[sl] = dlist[sl]
        cp1, cp2 = _gathers()
        cp1.start()
        cp2.start()
        _reset_lists()
        return jnp.int32(0), jnp.int32(1)

    chunk0 = sid * _CHUNK

    def _do_pass(p):
        lo = p * _R
        r0 = sid * _SHARE
        off = 0
        while off < _SHARE:
            n = min(_ZROWS, _SHARE - off)
            zsrc = zbuf if n == _ZROWS else zbuf.at[pl.ds(0, n)]
            pltpu.sync_copy(zsrc, acc.at[pl.ds(r0 + off, n)])
            off += n
        plsc.subcore_barrier()

        def _blk(b, carry):
            t0 = chunk0 + b * _BSC
            pltpu.sync_copy(dst_h.at[pl.ds(t0, _BSC)], dstblk)
            pltpu.sync_copy(src_h.at[pl.ds(t0, _BSC)], srcblk)

            def _v(v, carry):
                cnt, pend = carry
                sl = pl.ds(v * 16, 16)
                d = dstblk[sl]
                s = srcblk[sl]
                dl = d - lo
                msk = (dl >= 0) & (dl < _R)
                mi = msk.astype(jnp.int32)
                tid = t0 + v * 16 + lax.iota(jnp.int32, 16)
                pos = jnp.where(msk, cnt + plsc.cumsum(mi) - 1, _GBUF - 1)
                dl = jnp.where(msk, dl, _R)
                plsc.store_scatter(slist, [pos], s)
                plsc.store_scatter(tlist, [pos], tid)
                plsc.store_scatter(dlist, [pos], dl)
                cnt = cnt + jnp.sum(mi)
                return lax.cond(cnt >= _GB, _fill, lambda p: (cnt, p), pend)
            return lax.fori_loop(0, _VPB, _v, carry)

        cnt, pend = lax.fori_loop(0, _NBLK, _blk,
                                  (jnp.int32(0), jnp.int32(0)))
        _, pend = lax.cond(cnt > 0, _fill, lambda p: (jnp.int32(0), p), pend)

        @pl.when(pend == 1)
        def _():
            _flush_pending()
        plsc.subcore_barrier()
        pltpu.sync_copy(acc.at[pl.ds(r0, _SHARE)], out_h.at[pl.ds(lo + r0, _SHARE)])
        plsc.subcore_barrier()

    def _pk(k, c):
        p = k * _NC + cid

        @pl.when(p < _NPASS)
        def _():
            _do_pass(p)
        return c
    lax.fori_loop(0, _NPASS // _NC, _pk, 0)


def _sc_segsum(src, dst, xkj, sbft):
    mesh = plsc.VectorSubcoreMesh(core_axis_name="c", subcore_axis_name="s")
    fn = functools.partial(
        pl.kernel,
        out_type=jax.ShapeDtypeStruct((_EPAD, INT), jnp.float32),
        mesh=mesh,
        scratch_types=[
            pltpu.VMEM((_BSC,), jnp.int32),
            pltpu.VMEM((_BSC,), jnp.int32),
            pltpu.VMEM((_GBUF,), jnp.int32),
            pltpu.VMEM((_GBUF,), jnp.int32),
            pltpu.VMEM((_GBUF,), jnp.int32),
            pltpu.VMEM((_GBUF,), jnp.int32),
            pltpu.VMEM((_GBUF,), jnp.int32),
            pltpu.VMEM((_GBUF,), jnp.int32),
            pltpu.VMEM((_GBUF, INT), jnp.float32),
            pltpu.VMEM((_GBUF, INT), jnp.float32),
            pltpu.VMEM((_ZROWS, INT), jnp.float32),
            pltpu.VMEM_SHARED((_ACC, INT), jnp.float32),
            pltpu.SemaphoreType.DMA,
            pltpu.SemaphoreType.DMA,
        ],
        compiler_params=pltpu.CompilerParams(
            needs_layout_passes=False,
            use_tc_tiling_on_sc=False,
        ),
    )(_sc_body)
    return fn(src, dst, xkj, sbft)


def kernel(m, rbf, sbf, triplet_index, W_rbf1, W_rbf2, W_sbf1, W_sbf2,
           W_ji, b_ji, W_kj, b_kj, W_down, W_up, Wb1, bb1, Wb2, bb2, Wf, bf,
           Wa1, ba1, Wa2, ba2):
    xkj = _edge_call(m, rbf, W_rbf1, W_rbf2, W_kj, b_kj, W_down)
    sbft = _sbf_call(sbf, W_sbf1, W_sbf2)
    src = triplet_index[0]
    dst = triplet_index[1]
    magg = _sc_segsum(src, dst, xkj, sbft)[:E]
    return _epi_call(m, magg, W_up, W_ji, b_ji, Wb1, bb1, Wb2, bb2, Wf, bf,
                     Wa1, ba1, Wa2, ba2)


# speedup vs baseline: 1.3089x; 1.0007x over previous
import functools

import jax
import jax.numpy as jnp
from jax import lax
from jax.experimental import pallas as pl
from jax.experimental.pallas import tpu as pltpu
from jax.experimental.pallas import tpu_sc as plsc

E = 320000
T = 1280000
EMB = 128
INT = 64

_NC, _NS = 2, 16
_R = 22912
_NPASS = 14
_EPAD = _R * _NPASS
_SHARE = _R // _NS
_ZROWS = 128
_CHUNK = T // _NS
_BSC = 800
_NBLK = _CHUNK // _BSC
_NPAIR = _NBLK // 2
_VPB = _BSC // 16
_GBUF = 128
_GB = 112
_ACC = _R + 8


def _edge_body(m_ref, rbf_ref, wr1, wr2, wkj, bkj, wdown, out_ref):
    rbf_t = (rbf_ref[...] @ wr1[...]) @ wr2[...]
    xkj = m_ref[...] @ wkj[...] + bkj[...]
    out_ref[...] = (xkj * rbf_t) @ wdown[...]


def _sbf_body(sbf_ref, ws1, ws2, out_ref):
    out_ref[...] = (sbf_ref[...] @ ws1[...]) @ ws2[...]


def _epi_body(m_ref, magg_ref, wup, wji, bji, wb1, bb1, wb2, bb2, wf, bf_,
              wa1, ba1, wa2, ba2, out_ref):
    m = m_ref[...]
    mu = magg_ref[...] @ wup[...] + (m @ wji[...] + bji[...])
    h = (mu @ wb1[...] + bb1[...]) @ wb2[...] + bb2[...]
    mu = mu + h
    mu = mu @ wf[...] + bf_[...]
    mo = m + mu
    a1 = wa1[...]
    a2 = wa2[...]
    c1 = ba1[...]
    c2 = ba2[...]
    for i in range(2):
        h = (mo @ a1[i] + c1[i]) @ a2[i] + c2[i]
        mo = mo + h
    out_ref[...] = mo


def _full(shape):
    nd = len(shape)
    return pl.BlockSpec(shape, lambda i: (0,) * nd)


def _edge_call(m, rbf, wr1, wr2, wkj, bkj, wdown):
    be = 2560
    grid = (E // be,)
    return pl.pallas_call(
        _edge_body,
        grid=grid,
        in_specs=[
            pl.BlockSpec((be, EMB), lambda i: (i, 0)),
            pl.BlockSpec((be, 6), lambda i: (i, 0)),
            _full(wr1.shape), _full(wr2.shape), _full(wkj.shape),
            _full(bkj.shape), _full(wdown.shape),
        ],
        out_specs=pl.BlockSpec((be, INT), lambda i: (i, 0)),
        out_shape=jax.ShapeDtypeStruct((E, INT), jnp.float32),
    )(m, rbf, wr1, wr2, wkj, bkj, wdown)


def _sbf_call(sbf, ws1, ws2):
    bt = 5120
    grid = (T // bt,)
    return pl.pallas_call(
        _sbf_body,
        grid=grid,
        in_specs=[
            pl.BlockSpec((bt, 42), lambda i: (i, 0)),
            _full(ws1.shape), _full(ws2.shape),
        ],
        out_specs=pl.BlockSpec((bt, INT), lambda i: (i, 0)),
        out_shape=jax.ShapeDtypeStruct((T, INT), jnp.float32),
    )(sbf, ws1, ws2)


def _epi_call(m, magg, wup, wji, bji, wb1, bb1, wb2, bb2, wf, bf_, wa1, ba1, wa2, ba2):
    be = 2560
    grid = (E // be,)
    args = (wup, wji, bji, wb1, bb1, wb2, bb2, wf, bf_, wa1, ba1, wa2, ba2)
    return pl.pallas_call(
        _epi_body,
        grid=grid,
        in_specs=[
            pl.BlockSpec((be, EMB), lambda i: (i, 0)),
            pl.BlockSpec((be, INT), lambda i: (i, 0)),
        ] + [_full(a.shape) for a in args],
        out_specs=pl.BlockSpec((be, EMB), lambda i: (i, 0)),
        out_shape=jax.ShapeDtypeStruct((E, EMB), jnp.float32),
    )(m, magg, *args)


def _sc_body(src_h, dst_h, xkj_h, sbft_h, out_h,
             dstblk0, srcblk0, dstblk1, srcblk1,
             slist, tlist, dlist, sslist, stlist, sdlist,
             xrows, srows, zbuf, acc, sem1, sem2,
             semd0, semc0, semd1, semc1):
    cid = lax.axis_index("c")
    sid = lax.axis_index("s")
    zf = jnp.zeros((16,), jnp.float32)
    zi = jnp.zeros((16,), jnp.int32)
    dummy = jnp.full((16,), _R, jnp.int32)
    iota16 = lax.iota(jnp.int32, 16)

    def _zb(i, c):
        for q in range(INT // 16):
            zbuf[i, pl.ds(q * 16, 16)] = zf
        return c
    lax.fori_loop(0, _ZROWS, _zb, 0)

    def _reset_lists():
        for j in range(_GBUF // 16):
            sl = pl.ds(j * 16, 16)
            slist[sl] = zi
            tlist[sl] = zi
            dlist[sl] = dummy
    _reset_lists()

    def _gathers():
        cp1 = pltpu.make_async_copy(xkj_h.at[sslist], xrows, sem1)
        cp2 = pltpu.make_async_copy(sbft_h.at[stlist], srows, sem2)
        return cp1, cp2

    def _flush_pending():
        cp1, cp2 = _gathers()
        cp1.wait()
        cp2.wait()

        def _mul(r, c):
            for q in range(INT // 16):
                sl = pl.ds(q * 16, 16)
                srows[r, sl] = srows[r, sl] * xrows[r, sl]
            return c
        lax.fori_loop(0, _GBUF, _mul, 0)
        pltpu.sync_copy(srows, acc.at[sdlist], add=True)

    def _fill(pend):
        @pl.when(pend == 1)
        def _():
            _flush_pending()
        for j in range(_GBUF // 16):
            sl = pl.ds(j * 16, 16)
            sslist[sl] = slist[sl]
            stlist[sl] = tlist[sl]
            sdlist[sl] = dlist[sl]
        cp1, cp2 = _gathers()
        cp1.start()
        cp2.start()
        _reset_lists()
        return jnp.int32(0), jnp.int32(1)

    chunk0 = sid * _CHUNK

    def _do_pass(p):
        lo = p * _R
        r0 = sid * _SHARE
        off = 0
        while off < _SHARE:
            n = min(_ZROWS, _SHARE - off)
            zsrc = zbuf if n == _ZROWS else zbuf.at[pl.ds(0, n)]
            pltpu.sync_copy(zsrc, acc.at[pl.ds(r0 + off, n)])
            off += n
        plsc.subcore_barrier()

        def _scan_block(dblk, sblk, t0, carry):
            def _v(v, carry):
                cnt, pend = carry
                sl = pl.ds(v * 16, 16)
                d = dblk[sl]
                s = sblk[sl]
                dl = d - lo
                msk = (dl >= 0) & (dl < _R)
                mi = msk.astype(jnp.int32)
                tid = t0 + v * 16 + iota16
                pos = jnp.where(msk, cnt + plsc.cumsum(mi) - 1, _GBUF - 1)
                dl = jnp.where(msk, dl, _R)
                plsc.store_scatter(slist, [pos], s)
                plsc.store_scatter(tlist, [pos], tid)
                plsc.store_scatter(dlist, [pos], dl)
                cnt = cnt + jnp.sum(mi)
                return lax.cond(cnt >= _GB, _fill, lambda p: (cnt, p), pend)
            return lax.fori_loop(0, _VPB, _v, carry)

        def _ld(t0, dblk, sblk, semd, semc):
            cpd = pltpu.make_async_copy(dst_h.at[pl.ds(t0, _BSC)], dblk, semd)
            cpc = pltpu.make_async_copy(src_h.at[pl.ds(t0, _BSC)], sblk, semc)
            return cpd, cpc

        cpd, cpc = _ld(chunk0, dstblk0, srcblk0, semd0, semc0)
        cpd.start()
        cpc.start()

        def _pair(k, carry):
            t0 = chunk0 + (2 * k) * _BSC
            cpd, cpc = _ld(t0, dstblk0, srcblk0, semd0, semc0)
            cpd.wait()
            cpc.wait()
            cpd, cpc = _ld(t0 + _BSC, dstblk1, srcblk1, semd1, semc1)
            cpd.start()
            cpc.start()
            carry = _scan_block(dstblk0, srcblk0, t0, carry)
            cpd, cpc = _ld(t0 + _BSC, dstblk1, srcblk1, semd1, semc1)
            cpd.wait()
            cpc.wait()

            @pl.when(k < _NPAIR - 1)
            def _():
                cpd, cpc = _ld(t0 + 2 * _BSC, dstblk0, srcblk0, semd0, semc0)
                cpd.start()
                cpc.start()
            return _scan_block(dstblk1, srcblk1, t0 + _BSC, carry)

        cnt, pend = lax.fori_loop(0, _NPAIR, _pair,
                                  (jnp.int32(0), jnp.int32(0)))
        _, pend = lax.cond(cnt > 0, _fill, lambda p: (jnp.int32(0), p), pend)

        @pl.when(pend == 1)
        def _():
            _flush_pending()
        plsc.subcore_barrier()
        pltpu.sync_copy(acc.at[pl.ds(r0, _SHARE)], out_h.at[pl.ds(lo + r0, _SHARE)])
        plsc.subcore_barrier()

    def _pk(k, c):
        p = k * _NC + cid

        @pl.when(p < _NPASS)
        def _():
            _do_pass(p)
        return c
    lax.fori_loop(0, _NPASS // _NC, _pk, 0)


def _sc_segsum(src, dst, xkj, sbft):
    mesh = plsc.VectorSubcoreMesh(core_axis_name="c", subcore_axis_name="s")
    fn = functools.partial(
        pl.kernel,
        out_type=jax.ShapeDtypeStruct((_EPAD, INT), jnp.float32),
        mesh=mesh,
        scratch_types=[
            pltpu.VMEM((_BSC,), jnp.int32),
            pltpu.VMEM((_BSC,), jnp.int32),
            pltpu.VMEM((_BSC,), jnp.int32),
            pltpu.VMEM((_BSC,), jnp.int32),
            pltpu.VMEM((_GBUF,), jnp.int32),
            pltpu.VMEM((_GBUF,), jnp.int32),
            pltpu.VMEM((_GBUF,), jnp.int32),
            pltpu.VMEM((_GBUF,), jnp.int32),
            pltpu.VMEM((_GBUF,), jnp.int32),
            pltpu.VMEM((_GBUF,), jnp.int32),
            pltpu.VMEM((_GBUF, INT), jnp.float32),
            pltpu.VMEM((_GBUF, INT), jnp.float32),
            pltpu.VMEM((_ZROWS, INT), jnp.float32),
            pltpu.VMEM_SHARED((_ACC, INT), jnp.float32),
            pltpu.SemaphoreType.DMA,
            pltpu.SemaphoreType.DMA,
            pltpu.SemaphoreType.DMA,
            pltpu.SemaphoreType.DMA,
            pltpu.SemaphoreType.DMA,
            pltpu.SemaphoreType.DMA,
        ],
        compiler_params=pltpu.CompilerParams(
            needs_layout_passes=False,
            use_tc_tiling_on_sc=False,
        ),
    )(_sc_body)
    return fn(src, dst, xkj, sbft)


def kernel(m, rbf, sbf, triplet_index, W_rbf1, W_rbf2, W_sbf1, W_sbf2,
           W_ji, b_ji, W_kj, b_kj, W_down, W_up, Wb1, bb1, Wb2, bb2, Wf, bf,
           Wa1, ba1, Wa2, ba2):
    xkj = _edge_call(m, rbf, W_rbf1, W_rbf2, W_kj, b_kj, W_down)
    sbft = _sbf_call(sbf, W_sbf1, W_sbf2)
    src = triplet_index[0]
    dst = triplet_index[1]
    magg = _sc_segsum(src, dst, xkj, sbft)[:E]
    return _epi_call(m, magg, W_up, W_ji, b_ji, Wb1, bb1, Wb2, bb2, Wf, bf,
                     Wa1, ba1, Wa2, ba2)


# speedup vs baseline: 1.3344x; 1.0195x over previous
import functools

import jax
import jax.numpy as jnp
from jax import lax
from jax.experimental import pallas as pl
from jax.experimental.pallas import tpu as pltpu
from jax.experimental.pallas import tpu_sc as plsc

E = 320000
T = 1280000
EMB = 128
INT = 64

_NC, _NS = 2, 16
_R = 22912
_NPASS = 14
_EPAD = _R * _NPASS
_SHARE = _R // _NS
_ZROWS = 128
_CHUNK = T // _NS
_BSC = 800
_NBLK = _CHUNK // _BSC
_NPAIR = _NBLK // 2
_VPB = _BSC // 16
_GBUF = 128
_GB = 112
_ACC = _R + 8


def _edge_body(m_ref, rbf_ref, wr1, wr2, wkj, bkj, wdown, out_ref):
    rbf_t = (rbf_ref[...] @ wr1[...]) @ wr2[...]
    xkj = m_ref[...] @ wkj[...] + bkj[...]
    out_ref[...] = (xkj * rbf_t) @ wdown[...]


def _sbf_body(sbf_ref, ws1, ws2, out_ref):
    out_ref[...] = (sbf_ref[...] @ ws1[...]) @ ws2[...]


def _epi_body(m_ref, magg_ref, a_ref, b_ref, c_ref, out_ref):
    out_ref[...] = m_ref[...] @ a_ref[...] + magg_ref[...] @ b_ref[...] + c_ref[...]


def _full(shape):
    nd = len(shape)
    return pl.BlockSpec(shape, lambda i: (0,) * nd)


def _edge_call(m, rbf, wr1, wr2, wkj, bkj, wdown):
    be = 2560
    grid = (E // be,)
    return pl.pallas_call(
        _edge_body,
        grid=grid,
        in_specs=[
            pl.BlockSpec((be, EMB), lambda i: (i, 0)),
            pl.BlockSpec((be, 6), lambda i: (i, 0)),
            _full(wr1.shape), _full(wr2.shape), _full(wkj.shape),
            _full(bkj.shape), _full(wdown.shape),
        ],
        out_specs=pl.BlockSpec((be, INT), lambda i: (i, 0)),
        out_shape=jax.ShapeDtypeStruct((E, INT), jnp.float32),
    )(m, rbf, wr1, wr2, wkj, bkj, wdown)


def _sbf_call(sbf, ws1, ws2):
    bt = 5120
    grid = (T // bt,)
    return pl.pallas_call(
        _sbf_body,
        grid=grid,
        in_specs=[
            pl.BlockSpec((bt, 42), lambda i: (i, 0)),
            _full(ws1.shape), _full(ws2.shape),
        ],
        out_specs=pl.BlockSpec((bt, INT), lambda i: (i, 0)),
        out_shape=jax.ShapeDtypeStruct((T, INT), jnp.float32),
    )(sbf, ws1, ws2)


def _epi_call(m, magg, a, b, c):
    be = 2560
    grid = (E // be,)
    return pl.pallas_call(
        _epi_body,
        grid=grid,
        in_specs=[
            pl.BlockSpec((be, EMB), lambda i: (i, 0)),
            pl.BlockSpec((be, INT), lambda i: (i, 0)),
            _full(a.shape), _full(b.shape), _full(c.shape),
        ],
        out_specs=pl.BlockSpec((be, EMB), lambda i: (i, 0)),
        out_shape=jax.ShapeDtypeStruct((E, EMB), jnp.float32),
    )(m, magg, a, b, c)


def _sc_body(src_h, dst_h, xkj_h, sbft_h, out_h,
             dstblk0, srcblk0, dstblk1, srcblk1,
             slist, tlist, dlist, sslist, stlist, sdlist,
             xrows, srows, zbuf, acc, sem1, sem2,
             semd0, semc0, semd1, semc1):
    cid = lax.axis_index("c")
    sid = lax.axis_index("s")
    zf = jnp.zeros((16,), jnp.float32)
    zi = jnp.zeros((16,), jnp.int32)
    dummy = jnp.full((16,), _R, jnp.int32)
    iota16 = lax.iota(jnp.int32, 16)

    def _zb(i, c):
        for q in range(INT // 16):
            zbuf[i, pl.ds(q * 16, 16)] = zf
        return c
    lax.fori_loop(0, _ZROWS, _zb, 0)

    def _reset_lists():
        for j in range(_GBUF // 16):
            sl = pl.ds(j * 16, 16)
            slist[sl] = zi
            tlist[sl] = zi
            dlist[sl] = dummy
    _reset_lists()

    def _gathers():
        cp1 = pltpu.make_async_copy(xkj_h.at[sslist], xrows, sem1)
        cp2 = pltpu.make_async_copy(sbft_h.at[stlist], srows, sem2)
        return cp1, cp2

    def _flush_pending():
        cp1, cp2 = _gathers()
        cp1.wait()
        cp2.wait()

        def _mul(r, c):
            for q in range(INT // 16):
                sl = pl.ds(q * 16, 16)
                srows[r, sl] = srows[r, sl] * xrows[r, sl]
            return c
        lax.fori_loop(0, _GBUF, _mul, 0)
        pltpu.sync_copy(srows, acc.at[sdlist], add=True)

    def _fill(pend):
        @pl.when(pend == 1)
        def _():
            _flush_pending()
        for j in range(_GBUF // 16):
            sl = pl.ds(j * 16, 16)
            sslist[sl] = slist[sl]
            stlist[sl] = tlist[sl]
            sdlist[sl] = dlist[sl]
        cp1, cp2 = _gathers()
        cp1.start()
        cp2.start()
        _reset_lists()
        return jnp.int32(0), jnp.int32(1)

    chunk0 = sid * _CHUNK

    def _do_pass(p):
        lo = p * _R
        r0 = sid * _SHARE
        off = 0
        while off < _SHARE:
            n = min(_ZROWS, _SHARE - off)
            zsrc = zbuf if n == _ZROWS else zbuf.at[pl.ds(0, n)]
            pltpu.sync_copy(zsrc, acc.at[pl.ds(r0 + off, n)])
            off += n
        plsc.subcore_barrier()

        def _scan_block(dblk, sblk, t0, carry):
            def _v(v, carry):
                cnt, pend = carry
                sl = pl.ds(v * 16, 16)
                d = dblk[sl]
                s = sblk[sl]
                dl = d - lo
                msk = (dl >= 0) & (dl < _R)
                mi = msk.astype(jnp.int32)
                tid = t0 + v * 16 + iota16
                pos = jnp.where(msk, cnt + plsc.cumsum(mi) - 1, _GBUF - 1)
                dl = jnp.where(msk, dl, _R)
                plsc.store_scatter(slist, [pos], s)
                plsc.store_scatter(tlist, [pos], tid)
                plsc.store_scatter(dlist, [pos], dl)
                cnt = cnt + jnp.sum(mi)
                return lax.cond(cnt >= _GB, _fill, lambda p: (cnt, p), pend)
            return lax.fori_loop(0, _VPB, _v, carry)

        def _ld(t0, dblk, sblk, semd, semc):
            cpd = pltpu.make_async_copy(dst_h.at[pl.ds(t0, _BSC)], dblk, semd)
            cpc = pltpu.make_async_copy(src_h.at[pl.ds(t0, _BSC)], sblk, semc)
            return cpd, cpc

        cpd, cpc = _ld(chunk0, dstblk0, srcblk0, semd0, semc0)
        cpd.start()
        cpc.start()

        def _pair(k, carry):
            t0 = chunk0 + (2 * k) * _BSC
            cpd, cpc = _ld(t0, dstblk0, srcblk0, semd0, semc0)
            cpd.wait()
            cpc.wait()
            cpd, cpc = _ld(t0 + _BSC, dstblk1, srcblk1, semd1, semc1)
            cpd.start()
            cpc.start()
            carry = _scan_block(dstblk0, srcblk0, t0, carry)
            cpd, cpc = _ld(t0 + _BSC, dstblk1, srcblk1, semd1, semc1)
            cpd.wait()
            cpc.wait()

            @pl.when(k < _NPAIR - 1)
            def _():
                cpd, cpc = _ld(t0 + 2 * _BSC, dstblk0, srcblk0, semd0, semc0)
                cpd.start()
                cpc.start()
            return _scan_block(dstblk1, srcblk1, t0 + _BSC, carry)

        cnt, pend = lax.fori_loop(0, _NPAIR, _pair,
                                  (jnp.int32(0), jnp.int32(0)))
        _, pend = lax.cond(cnt > 0, _fill, lambda p: (jnp.int32(0), p), pend)

        @pl.when(pend == 1)
        def _():
            _flush_pending()
        plsc.subcore_barrier()
        pltpu.sync_copy(acc.at[pl.ds(r0, _SHARE)], out_h.at[pl.ds(lo + r0, _SHARE)])
        plsc.subcore_barrier()

    def _pk(k, c):
        p = k * _NC + cid

        @pl.when(p < _NPASS)
        def _():
            _do_pass(p)
        return c
    lax.fori_loop(0, _NPASS // _NC, _pk, 0)


def _sc_segsum(src, dst, xkj, sbft):
    mesh = plsc.VectorSubcoreMesh(core_axis_name="c", subcore_axis_name="s")
    fn = functools.partial(
        pl.kernel,
        out_type=jax.ShapeDtypeStruct((_EPAD, INT), jnp.float32),
        mesh=mesh,
        scratch_types=[
            pltpu.VMEM((_BSC,), jnp.int32),
            pltpu.VMEM((_BSC,), jnp.int32),
            pltpu.VMEM((_BSC,), jnp.int32),
            pltpu.VMEM((_BSC,), jnp.int32),
            pltpu.VMEM((_GBUF,), jnp.int32),
            pltpu.VMEM((_GBUF,), jnp.int32),
            pltpu.VMEM((_GBUF,), jnp.int32),
            pltpu.VMEM((_GBUF,), jnp.int32),
            pltpu.VMEM((_GBUF,), jnp.int32),
            pltpu.VMEM((_GBUF,), jnp.int32),
            pltpu.VMEM((_GBUF, INT), jnp.float32),
            pltpu.VMEM((_GBUF, INT), jnp.float32),
            pltpu.VMEM((_ZROWS, INT), jnp.float32),
            pltpu.VMEM_SHARED((_ACC, INT), jnp.float32),
            pltpu.SemaphoreType.DMA,
            pltpu.SemaphoreType.DMA,
            pltpu.SemaphoreType.DMA,
            pltpu.SemaphoreType.DMA,
            pltpu.SemaphoreType.DMA,
            pltpu.SemaphoreType.DMA,
        ],
        compiler_params=pltpu.CompilerParams(
            needs_layout_passes=False,
            use_tc_tiling_on_sc=False,
        ),
    )(_sc_body)
    return fn(src, dst, xkj, sbft)


def kernel(m, rbf, sbf, triplet_index, W_rbf1, W_rbf2, W_sbf1, W_sbf2,
           W_ji, b_ji, W_kj, b_kj, W_down, W_up, Wb1, bb1, Wb2, bb2, Wf, bf,
           Wa1, ba1, Wa2, ba2):
    xkj = _edge_call(m, rbf, W_rbf1, W_rbf2, W_kj, b_kj, W_down)
    sbft = _sbf_call(sbf, W_sbf1, W_sbf2)
    src = triplet_index[0]
    dst = triplet_index[1]
    magg = _sc_segsum(src, dst, xkj, sbft)[:E]
    eye = jnp.eye(EMB, dtype=jnp.float32)
    mm = functools.partial(jnp.matmul, precision=lax.Precision.HIGHEST)
    s_ = mm(eye + mm(Wb1, Wb2), Wf)
    c_s = mm(mm(bb1, Wb2) + bb2, Wf) + bf
    r0 = eye + mm(Wa1[0], Wa2[0])
    r1 = eye + mm(Wa1[1], Wa2[1])
    c0 = mm(ba1[0], Wa2[0]) + ba2[0]
    c1 = mm(ba1[1], Wa2[1]) + ba2[1]
    r01 = mm(r0, r1)
    a = mm(eye + mm(W_ji, s_), r01)
    b = mm(mm(W_up, s_), r01)
    c = mm(mm(b_ji, s_) + c_s, r01) + mm(c0, r1) + c1
    return _epi_call(m, magg, a, b, c)
